# Initial kernel scaffold; baseline (speedup 1.0000x reference)
import jax
import jax.numpy as jnp
from jax.experimental import pallas as pl


def kernel(x, W_g):
    raise NotImplementedError("write your pallas kernel here")


# speedup vs baseline: 1.7773x; 1.7773x over previous
import functools

import jax
import jax.numpy as jnp
from jax.experimental import pallas as pl

_NUM_EXPERTS = 64
_BLOCK_M = 512


def _gating_body(x_ref, w_ref, o_ref):
    x = x_ref[...]
    w = w_ref[...]
    logits = jax.lax.dot_general(
        x, w, (((1,), (1,)), ((), ())), preferred_element_type=jnp.float32
    )
    v1 = jnp.max(logits, axis=-1, keepdims=True)
    iota = jax.lax.broadcasted_iota(jnp.int32, logits.shape, 1)
    idx1 = jnp.min(
        jnp.where(logits == v1, iota, _NUM_EXPERTS), axis=-1, keepdims=True
    )
    masked = jnp.where(iota == idx1, -jnp.inf, logits)
    v2 = jnp.max(masked, axis=-1, keepdims=True)
    e2 = jnp.exp(v2 - v1)
    denom = 1.0 + e2
    o_ref[...] = jnp.concatenate([1.0 / denom, e2 / denom], axis=-1)


@functools.partial(jax.jit, static_argnames=("interpret",))
def kernel(x, W_g, interpret=False):
    n_tokens, dim = x.shape
    grid = (n_tokens // _BLOCK_M,)
    return pl.pallas_call(
        _gating_body,
        grid=grid,
        in_specs=[
            pl.BlockSpec((_BLOCK_M, dim), lambda i: (i, 0)),
            pl.BlockSpec((_NUM_EXPERTS, dim), lambda i: (0, 0)),
        ],
        out_specs=pl.BlockSpec((_BLOCK_M, 2), lambda i: (i, 0)),
        out_shape=jax.ShapeDtypeStruct((n_tokens, 2), jnp.float32),
        interpret=interpret,
    )(x, W_g)


# speedup vs baseline: 2.1207x; 1.1932x over previous
import functools

import jax
import jax.numpy as jnp
from jax.experimental import pallas as pl

_NUM_EXPERTS = 64
_BLOCK_M = 1024


def _gating_body(x_ref, w_ref, o_ref):
    x = x_ref[...]
    w = w_ref[...]
    logits = jax.lax.dot_general(
        x, w, (((1,), (1,)), ((), ())), preferred_element_type=jnp.float32
    )
    v1 = jnp.max(logits, axis=-1, keepdims=True)
    iota = jax.lax.broadcasted_iota(jnp.int32, logits.shape, 1)
    idx1 = jnp.min(
        jnp.where(logits == v1, iota, _NUM_EXPERTS), axis=-1, keepdims=True
    )
    masked = jnp.where(iota == idx1, -jnp.inf, logits)
    v2 = jnp.max(masked, axis=-1, keepdims=True)
    e2 = jnp.exp(v2 - v1)
    denom = 1.0 + e2
    o_ref[...] = jnp.concatenate([1.0 / denom, e2 / denom], axis=-1)


@functools.partial(jax.jit, static_argnames=("interpret",))
def kernel(x, W_g, interpret=False):
    n_tokens, dim = x.shape
    grid = (n_tokens // _BLOCK_M,)
    return pl.pallas_call(
        _gating_body,
        grid=grid,
        in_specs=[
            pl.BlockSpec((_BLOCK_M, dim), lambda i: (i, 0)),
            pl.BlockSpec((_NUM_EXPERTS, dim), lambda i: (0, 0)),
        ],
        out_specs=pl.BlockSpec((_BLOCK_M, 2), lambda i: (i, 0)),
        out_shape=jax.ShapeDtypeStruct((n_tokens, 2), jnp.float32),
        interpret=interpret,
    )(x, W_g)


# speedup vs baseline: 2.1648x; 1.0208x over previous
import functools

import jax
import jax.numpy as jnp
from jax.experimental import pallas as pl

_NUM_EXPERTS = 64
_BLOCK_M = 2048


def _gating_body(x_ref, w_ref, o_ref):
    x = x_ref[...]
    w = w_ref[...]
    logits = jax.lax.dot_general(
        x, w, (((1,), (1,)), ((), ())), preferred_element_type=jnp.float32
    )
    v1 = jnp.max(logits, axis=-1, keepdims=True)
    iota = jax.lax.broadcasted_iota(jnp.int32, logits.shape, 1)
    idx1 = jnp.min(
        jnp.where(logits == v1, iota, _NUM_EXPERTS), axis=-1, keepdims=True
    )
    masked = jnp.where(iota == idx1, -jnp.inf, logits)
    v2 = jnp.max(masked, axis=-1, keepdims=True)
    e2 = jnp.exp(v2 - v1)
    denom = 1.0 + e2
    o_ref[...] = jnp.concatenate([1.0 / denom, e2 / denom], axis=-1)


@functools.partial(jax.jit, static_argnames=("interpret",))
def kernel(x, W_g, interpret=False):
    n_tokens, dim = x.shape
    grid = (n_tokens // _BLOCK_M,)
    return pl.pallas_call(
        _gating_body,
        grid=grid,
        in_specs=[
            pl.BlockSpec((_BLOCK_M, dim), lambda i: (i, 0)),
            pl.BlockSpec((_NUM_EXPERTS, dim), lambda i: (0, 0)),
        ],
        out_specs=pl.BlockSpec((_BLOCK_M, 2), lambda i: (i, 0)),
        out_shape=jax.ShapeDtypeStruct((n_tokens, 2), jnp.float32),
        interpret=interpret,
    )(x, W_g)
